# baseline (speedup 1.0000x reference)
import functools
import jax
import jax.numpy as jnp
from jax import lax
from jax.experimental import pallas as pl
from jax.experimental.pallas import tpu as pltpu
from jax.experimental.pallas import tpu_sc as plsc

B = 16384
D = 64
V = 1000000
L = 16
CHUNK = 128
FIRE = 16

_info = plsc.get_sparse_core_info()
NC = _info.num_cores
NS = _info.num_subcores
NW = NC * NS
B_PER_W = B // NW
N_CHUNKS = B_PER_W // CHUNK

_PARAMS = pltpu.CompilerParams(
    needs_layout_passes=False, use_tc_tiling_on_sc=False)
_MESH = dict(core_axis_name="c", subcore_axis_name="s")


def _gather_kernel(idx_hbm, vt, stage_hbm, vidx, gath, sem):
    wid = lax.axis_index("s") * NC + lax.axis_index("c")
    base = wid * B_PER_W

    pltpu.sync_copy(idx_hbm.at[pl.ds(wid * N_CHUNKS, N_CHUNKS)], vidx)

    for j in range(N_CHUNKS):
        def dloop(db, _):
            copies = []
            for t in range(FIRE):
                d = db * FIRE + t
                copies.append(pltpu.async_copy(
                    vt.at[d].at[vidx.at[j]],
                    gath.at[d, pl.ds(j * CHUNK, CHUNK)], sem))
            for cp in copies:
                cp.wait()
            return 0

        lax.fori_loop(0, D // FIRE, dloop, 0)

    pltpu.sync_copy(gath, stage_hbm.at[:, pl.ds(base, B_PER_W)])


def _combine_kernel(sa_hbm, sb_hbm, dot_hbm, ga, gb, out_v, sem):
    wid = lax.axis_index("s") * NC + lax.axis_index("c")
    base = wid * B_PER_W

    ca = pltpu.async_copy(sa_hbm.at[:, pl.ds(base, B_PER_W)], ga, sem)
    cb = pltpu.async_copy(sb_hbm.at[:, pl.ds(base, B_PER_W)], gb, sem)
    ca.wait()
    cb.wait()

    def dots(g, _):
        s = pl.ds(g * L, L)
        acc = None
        for d in range(D):
            prod = ga[d, s] * gb[d, s]
            acc = prod if acc is None else acc + prod
        out_v[s] = acc
        return 0

    lax.fori_loop(0, B_PER_W // L, dots, 0)

    pltpu.sync_copy(out_v, dot_hbm.at[pl.ds(base, B_PER_W)])


@jax.jit
def _word2vec_dot(center2d, context2d, vta, vtb):
    gather = functools.partial(
        pl.kernel,
        out_type=jax.ShapeDtypeStruct((D, B), jnp.float32),
        mesh=plsc.VectorSubcoreMesh(**_MESH),
        compiler_params=_PARAMS,
        scratch_types=[
            pltpu.VMEM((N_CHUNKS, CHUNK), jnp.int32),
            pltpu.VMEM((D, B_PER_W), jnp.float32),
            pltpu.SemaphoreType.DMA,
        ],
    )(_gather_kernel)
    sa = gather(center2d, vta)
    sb = gather(context2d, vtb)

    combine = functools.partial(
        pl.kernel,
        out_type=jax.ShapeDtypeStruct((B,), jnp.float32),
        mesh=plsc.VectorSubcoreMesh(**_MESH),
        compiler_params=_PARAMS,
        scratch_types=[
            pltpu.VMEM((D, B_PER_W), jnp.float32),
            pltpu.VMEM((D, B_PER_W), jnp.float32),
            pltpu.VMEM((B_PER_W,), jnp.float32),
            pltpu.SemaphoreType.DMA,
        ],
    )(_combine_kernel)
    return combine(sa, sb)


def kernel(center, context, in_table, out_table):
    center2d = center.astype(jnp.int32).reshape(NW * N_CHUNKS, CHUNK)
    context2d = context.astype(jnp.int32).reshape(NW * N_CHUNKS, CHUNK)
    return _word2vec_dot(center2d, context2d, in_table.T, out_table.T)


# speedup vs baseline: 8.9898x; 8.9898x over previous
import functools
import jax
import jax.numpy as jnp
from jax import lax
from jax.experimental import pallas as pl
from jax.experimental.pallas import tpu as pltpu
from jax.experimental.pallas import tpu_sc as plsc

B = 16384
D = 64
L = 16
CHUNK = 128

_info = plsc.get_sparse_core_info()
NC = _info.num_cores
NS = _info.num_subcores
NW = NC * NS
B_PER_W = B // NW
N_CHUNKS = B_PER_W // CHUNK
GROUPS = B_PER_W // L

_PARAMS = pltpu.CompilerParams(needs_layout_passes=False)
_MESH = dict(core_axis_name="c", subcore_axis_name="s")


def _gather_kernel(idx_hbm, tab2, stage_hbm, vidx, vrow, pairs, out_v, sem):
    wid = lax.axis_index("s") * NC + lax.axis_index("c")
    base = wid * B_PER_W
    lane = lax.iota(jnp.int32, L)

    pltpu.sync_copy(idx_hbm.at[pl.ds(wid * N_CHUNKS, N_CHUNKS)], vidx)

    def rows(j, _):
        def rows16(g, _):
            s = pl.ds(g * L, L)
            vrow[j, s] = lax.shift_right_logical(vidx[j, s], 1)
            return 0
        lax.fori_loop(0, CHUNK // L, rows16, 0)
        return 0

    lax.fori_loop(0, N_CHUNKS, rows, 0)

    copies = []
    for j in range(N_CHUNKS):
        copies.append(pltpu.async_copy(
            tab2.at[vrow.at[j]], pairs.at[pl.ds(j * CHUNK, CHUNK), :], sem))
    for cp in copies:
        cp.wait()

    def extract(g, _):
        va = vidx[lax.shift_right_logical(g, 3),
                  pl.ds((g & 7) * L, L)]
        half = (va & 1) * D
        row = g * L + lane
        obase = row * D
        for dd in range(D):
            dl = (lane + dd) & (D - 1)
            vals = plsc.load_gather(pairs, [row, half + dl])
            plsc.store_scatter(out_v, [obase + dl], vals)
        return 0

    lax.fori_loop(0, GROUPS, extract, 0)

    pltpu.sync_copy(out_v, stage_hbm.at[pl.ds(base * D, B_PER_W * D)])


def _combine_kernel(sa_hbm, sb_hbm, dot_hbm, ga, gb, out_v, sem):
    wid = lax.axis_index("s") * NC + lax.axis_index("c")
    base = wid * B_PER_W
    lane = lax.iota(jnp.int32, L)

    ca = pltpu.async_copy(sa_hbm.at[pl.ds(base * D, B_PER_W * D)], ga, sem)
    cb = pltpu.async_copy(sb_hbm.at[pl.ds(base * D, B_PER_W * D)], gb, sem)
    ca.wait()
    cb.wait()

    def dots(g, _):
        obase = (g * L + lane) * D
        acc = None
        for dd in range(D):
            addr = obase + ((lane + dd) & (D - 1))
            prod = plsc.load_gather(ga, [addr]) * plsc.load_gather(gb, [addr])
            acc = prod if acc is None else acc + prod
        out_v[pl.ds(g * L, L)] = acc
        return 0

    lax.fori_loop(0, GROUPS, dots, 0)

    pltpu.sync_copy(out_v, dot_hbm.at[pl.ds(base, B_PER_W)])


@jax.jit
def _word2vec_dot(center2d, context2d, ta2, tb2):
    gather = functools.partial(
        pl.kernel,
        out_type=jax.ShapeDtypeStruct((B * D,), jnp.float32),
        mesh=plsc.VectorSubcoreMesh(**_MESH),
        compiler_params=_PARAMS,
        scratch_types=[
            pltpu.VMEM((N_CHUNKS, CHUNK), jnp.int32),
            pltpu.VMEM((N_CHUNKS, CHUNK), jnp.int32),
            pltpu.VMEM((B_PER_W, CHUNK), jnp.float32),
            pltpu.VMEM((B_PER_W * D,), jnp.float32),
            pltpu.SemaphoreType.DMA,
        ],
    )(_gather_kernel)
    sa = gather(center2d, ta2)
    sb = gather(context2d, tb2)

    combine = functools.partial(
        pl.kernel,
        out_type=jax.ShapeDtypeStruct((B,), jnp.float32),
        mesh=plsc.VectorSubcoreMesh(**_MESH),
        compiler_params=_PARAMS,
        scratch_types=[
            pltpu.VMEM((B_PER_W * D,), jnp.float32),
            pltpu.VMEM((B_PER_W * D,), jnp.float32),
            pltpu.VMEM((B_PER_W,), jnp.float32),
            pltpu.SemaphoreType.DMA,
        ],
    )(_combine_kernel)
    return combine(sa, sb)


def kernel(center, context, in_table, out_table):
    center2d = center.astype(jnp.int32).reshape(NW * N_CHUNKS, CHUNK)
    context2d = context.astype(jnp.int32).reshape(NW * N_CHUNKS, CHUNK)
    ta2 = in_table.reshape(500000, CHUNK)
    tb2 = out_table.reshape(500000, CHUNK)
    return _word2vec_dot(center2d, context2d, ta2, tb2)


# speedup vs baseline: 35.3092x; 3.9277x over previous
import functools
import jax
import jax.numpy as jnp
from jax import lax
from jax.experimental import pallas as pl
from jax.experimental.pallas import tpu as pltpu
from jax.experimental.pallas import tpu_sc as plsc

B = 16384
D = 64
V = 1000000
L = 16
BLK = 128
LAST_BLK = (V - 1) // BLK
WIN = 2
BIG = 1 << 30

_info = plsc.get_sparse_core_info()
NC = _info.num_cores
NS = _info.num_subcores
NW = NC * NS
B_PER_W = B // NW
GROUPS = B_PER_W // L

_PARAMS = pltpu.CompilerParams(needs_layout_passes=False)
_MESH = dict(core_axis_name="c", subcore_axis_name="s")


def _sweep_table(sv_hbm, sp_hbm, vt, stage_hbm, sv, sp, slab, colst, sem,
                 base, lane):
    pltpu.sync_copy(sv_hbm.at[pl.ds(base, B_PER_W)], sv.at[pl.ds(0, B_PER_W)])
    sv[pl.ds(B_PER_W, L)] = jnp.full((L,), BIG, jnp.int32)
    sv[pl.ds(B_PER_W + L, L)] = jnp.full((L,), BIG, jnp.int32)
    pltpu.sync_copy(sp_hbm.at[pl.ds(base, B_PER_W)], sp)

    blk_lo = lax.shift_right_logical(sv[pl.ds(0, L)][0], 7)
    blk_hi = lax.shift_right_logical(sv[pl.ds(B_PER_W - L, L)][L - 1], 7)
    nwin = lax.div(blk_hi - blk_lo, WIN) + 1
    nwp = lax.div(nwin + 1, 2)

    def fire(w, s):
        for b in range(WIN):
            blk = jnp.minimum(blk_lo + w * WIN + b, LAST_BLK)
            start = pl.multiple_of(blk * BLK, BLK)
            pltpu.async_copy(vt.at[:, pl.ds(start, BLK)], slab.at[s, b], sem)

    def drain(s):
        for b in range(WIN):
            pltpu.make_async_copy(
                vt.at[:, pl.ds(0, BLK)], slab.at[s, b], sem).wait()

    def process(w, s, p):
        wbase = blk_lo + w * WIN
        wend = (wbase + WIN) * BLK
        view = slab.at[s]

        def wcond(st):
            return st[1]

        def wbody(st):
            p, _ = st
            v16 = plsc.load_gather(sv, [p + lane])
            mask = v16 < wend
            cnt = plsc.all_reduce_population_count(mask)[0]
            q = lax.shift_right_logical(v16, 7) - wbase
            col = v16 & (BLK - 1)
            obase = (p + lane) * D
            for dd in range(D):
                rows = (lane + dd) & (D - 1)
                vals = plsc.load_gather(view, [q, rows, col], mask=mask)
                plsc.store_scatter(colst, [obase + rows], vals, mask=mask)
            return p + cnt, cnt == L

        p, _ = lax.while_loop(wcond, wbody, (p, True))
        return p

    fire(0, 0)

    def body(wp, p):
        w0 = wp * 2
        fire(w0 + 1, 1)
        drain(0)
        p = process(w0, 0, p)
        fire(w0 + 2, 0)
        drain(1)
        p = process(w0 + 1, 1, p)
        return p

    lax.fori_loop(0, nwp, body, jnp.int32(0))
    drain(0)

    def stage(g, _):
        pos16 = sp[pl.ds(g * L, L)]
        copies = []
        for t in range(L):
            src = colst.at[pl.ds((g * L + t) * D, D)]
            dst = stage_hbm.at[pl.ds(pos16[t] * D, D)]
            copies.append(pltpu.async_copy(src, dst, sem))
        for cp in copies:
            cp.wait()
        return 0

    lax.fori_loop(0, GROUPS, stage, 0)


def _gather_kernel(sva, spa, svb, spb, vta, vtb, sa_hbm, sb_hbm,
                   sv, sp, slab, colst, sem):
    wid = lax.axis_index("s") * NC + lax.axis_index("c")
    base = wid * B_PER_W
    lane = lax.iota(jnp.int32, L)
    _sweep_table(sva, spa, vta, sa_hbm, sv, sp, slab, colst, sem, base, lane)
    _sweep_table(svb, spb, vtb, sb_hbm, sv, sp, slab, colst, sem, base, lane)


def _combine_kernel(sa_hbm, sb_hbm, dot_hbm, ga, gb, out_v, sem):
    wid = lax.axis_index("s") * NC + lax.axis_index("c")
    base = wid * B_PER_W
    lane = lax.iota(jnp.int32, L)

    ca = pltpu.async_copy(sa_hbm.at[pl.ds(base * D, B_PER_W * D)], ga, sem)
    cb = pltpu.async_copy(sb_hbm.at[pl.ds(base * D, B_PER_W * D)], gb, sem)
    ca.wait()
    cb.wait()

    def dots(g, _):
        obase = (g * L + lane) * D
        acc = None
        for dd in range(D):
            addr = obase + ((lane + dd) & (D - 1))
            prod = plsc.load_gather(ga, [addr]) * plsc.load_gather(gb, [addr])
            acc = prod if acc is None else acc + prod
        out_v[pl.ds(g * L, L)] = acc
        return 0

    lax.fori_loop(0, GROUPS, dots, 0)

    pltpu.sync_copy(out_v, dot_hbm.at[pl.ds(base, B_PER_W)])


@jax.jit
def _word2vec_dot(sva, spa, svb, spb, vta, vtb):
    gather = functools.partial(
        pl.kernel,
        out_type=(jax.ShapeDtypeStruct((B * D,), jnp.float32),
                  jax.ShapeDtypeStruct((B * D,), jnp.float32)),
        mesh=plsc.VectorSubcoreMesh(**_MESH),
        compiler_params=_PARAMS,
        scratch_types=[
            pltpu.VMEM((B_PER_W + 2 * L,), jnp.int32),
            pltpu.VMEM((B_PER_W,), jnp.int32),
            pltpu.VMEM((2, WIN, D, BLK), jnp.float32),
            pltpu.VMEM((B_PER_W * D,), jnp.float32),
            pltpu.SemaphoreType.DMA,
        ],
    )(_gather_kernel)
    sa, sb = gather(sva, spa, svb, spb, vta, vtb)

    combine = functools.partial(
        pl.kernel,
        out_type=jax.ShapeDtypeStruct((B,), jnp.float32),
        mesh=plsc.VectorSubcoreMesh(**_MESH),
        compiler_params=_PARAMS,
        scratch_types=[
            pltpu.VMEM((B_PER_W * D,), jnp.float32),
            pltpu.VMEM((B_PER_W * D,), jnp.float32),
            pltpu.VMEM((B_PER_W,), jnp.float32),
            pltpu.SemaphoreType.DMA,
        ],
    )(_combine_kernel)
    return combine(sa, sb)


def kernel(center, context, in_table, out_table):
    c32 = center.astype(jnp.int32)
    x32 = context.astype(jnp.int32)
    iota = jnp.arange(B, dtype=jnp.int32)
    sva, spa = lax.sort_key_val(c32, iota)
    svb, spb = lax.sort_key_val(x32, iota)
    return _word2vec_dot(sva, spa, svb, spb, in_table.T, out_table.T)


# speedup vs baseline: 43.7716x; 1.2397x over previous
import functools
import jax
import jax.numpy as jnp
from jax import lax
from jax.experimental import pallas as pl
from jax.experimental.pallas import tpu as pltpu
from jax.experimental.pallas import tpu_sc as plsc

B = 16384
D = 64
V = 1000000
L = 16
BLK = 128
LAST_BLK = (V - 1) // BLK
WIN = 4
BIG = 1 << 30

_info = plsc.get_sparse_core_info()
NC = _info.num_cores
NS = _info.num_subcores
NW = NC * NS
B_PER_W = B // NW
GROUPS = B_PER_W // L

_PARAMS = pltpu.CompilerParams(needs_layout_passes=False)
_MESH = dict(core_axis_name="c", subcore_axis_name="s")


def _sweep_table(sv_hbm, sp_hbm, vt, stage_hbm, sv, sp, slab, colst, sem,
                 base, lane):
    pltpu.sync_copy(sv_hbm.at[pl.ds(base, B_PER_W)], sv.at[pl.ds(0, B_PER_W)])
    sv[pl.ds(B_PER_W, L)] = jnp.full((L,), BIG, jnp.int32)
    sv[pl.ds(B_PER_W + L, L)] = jnp.full((L,), BIG, jnp.int32)
    pltpu.sync_copy(sp_hbm.at[pl.ds(base, B_PER_W)], sp)

    blk_lo = lax.shift_right_logical(sv[pl.ds(0, L)][0], 7)
    blk_hi = lax.shift_right_logical(sv[pl.ds(B_PER_W - L, L)][L - 1], 7)
    nwin = lax.div(blk_hi - blk_lo, WIN) + 1
    nwp = lax.div(nwin + 1, 2)

    def fire(w, s):
        for b in range(WIN):
            blk = jnp.minimum(blk_lo + w * WIN + b, LAST_BLK)
            start = pl.multiple_of(blk * BLK, BLK)
            pltpu.async_copy(vt.at[:, pl.ds(start, BLK)], slab.at[s, b], sem)

    def drain(s):
        for b in range(WIN):
            pltpu.make_async_copy(
                vt.at[:, pl.ds(0, BLK)], slab.at[s, b], sem).wait()

    def process(w, s, p):
        wbase = blk_lo + w * WIN
        wend = (wbase + WIN) * BLK
        view = slab.at[s]

        def wcond(st):
            return st[1]

        def wbody(st):
            p, _ = st
            v16 = plsc.load_gather(sv, [p + lane])
            mask = v16 < wend
            cnt = plsc.all_reduce_population_count(mask)[0]
            q = lax.shift_right_logical(v16, 7) - wbase
            col = v16 & (BLK - 1)
            obase = (p + lane) * D
            for dd in range(D):
                rows = (lane + dd) & (D - 1)
                vals = plsc.load_gather(view, [q, rows, col], mask=mask)
                plsc.store_scatter(colst, [obase + rows], vals, mask=mask)
            return p + cnt, cnt == L

        p, _ = lax.while_loop(wcond, wbody, (p, True))
        return p

    fire(0, 0)

    def body(wp, p):
        w0 = wp * 2
        fire(w0 + 1, 1)
        drain(0)
        p = process(w0, 0, p)
        fire(w0 + 2, 0)
        drain(1)
        p = process(w0 + 1, 1, p)
        return p

    lax.fori_loop(0, nwp, body, jnp.int32(0))
    drain(0)

    def stage(g, _):
        pos16 = sp[pl.ds(g * L, L)]
        copies = []
        for t in range(L):
            src = colst.at[pl.ds((g * L + t) * D, D)]
            dst = stage_hbm.at[pl.ds(pos16[t] * D, D)]
            copies.append(pltpu.async_copy(src, dst, sem))
        for cp in copies:
            cp.wait()
        return 0

    lax.fori_loop(0, GROUPS, stage, 0)


def _gather_kernel(sva, spa, svb, spb, vta, vtb, sa_hbm, sb_hbm,
                   sv, sp, slab, colst, sem):
    wid = lax.axis_index("s") * NC + lax.axis_index("c")
    base = wid * B_PER_W
    lane = lax.iota(jnp.int32, L)
    _sweep_table(sva, spa, vta, sa_hbm, sv, sp, slab, colst, sem, base, lane)
    _sweep_table(svb, spb, vtb, sb_hbm, sv, sp, slab, colst, sem, base, lane)


def _combine_kernel(sa_hbm, sb_hbm, dot_hbm, ga, gb, out_v, sem):
    wid = lax.axis_index("s") * NC + lax.axis_index("c")
    base = wid * B_PER_W
    lane = lax.iota(jnp.int32, L)

    ca = pltpu.async_copy(sa_hbm.at[pl.ds(base * D, B_PER_W * D)], ga, sem)
    cb = pltpu.async_copy(sb_hbm.at[pl.ds(base * D, B_PER_W * D)], gb, sem)
    ca.wait()
    cb.wait()

    def dots(g, _):
        obase = (g * L + lane) * D
        acc = None
        for dd in range(D):
            addr = obase + ((lane + dd) & (D - 1))
            prod = plsc.load_gather(ga, [addr]) * plsc.load_gather(gb, [addr])
            acc = prod if acc is None else acc + prod
        out_v[pl.ds(g * L, L)] = acc
        return 0

    lax.fori_loop(0, GROUPS, dots, 0)

    pltpu.sync_copy(out_v, dot_hbm.at[pl.ds(base, B_PER_W)])


@jax.jit
def _word2vec_dot(sva, spa, svb, spb, vta, vtb):
    gather = functools.partial(
        pl.kernel,
        out_type=(jax.ShapeDtypeStruct((B * D,), jnp.float32),
                  jax.ShapeDtypeStruct((B * D,), jnp.float32)),
        mesh=plsc.VectorSubcoreMesh(**_MESH),
        compiler_params=_PARAMS,
        scratch_types=[
            pltpu.VMEM((B_PER_W + 2 * L,), jnp.int32),
            pltpu.VMEM((B_PER_W,), jnp.int32),
            pltpu.VMEM((2, WIN, D, BLK), jnp.float32),
            pltpu.VMEM((B_PER_W * D,), jnp.float32),
            pltpu.SemaphoreType.DMA,
        ],
    )(_gather_kernel)
    sa, sb = gather(sva, spa, svb, spb, vta, vtb)

    combine = functools.partial(
        pl.kernel,
        out_type=jax.ShapeDtypeStruct((B,), jnp.float32),
        mesh=plsc.VectorSubcoreMesh(**_MESH),
        compiler_params=_PARAMS,
        scratch_types=[
            pltpu.VMEM((B_PER_W * D,), jnp.float32),
            pltpu.VMEM((B_PER_W * D,), jnp.float32),
            pltpu.VMEM((B_PER_W,), jnp.float32),
            pltpu.SemaphoreType.DMA,
        ],
    )(_combine_kernel)
    return combine(sa, sb)


def kernel(center, context, in_table, out_table):
    c32 = center.astype(jnp.int32)
    x32 = context.astype(jnp.int32)
    iota = jnp.arange(B, dtype=jnp.int32)
    sva, spa = lax.sort_key_val(c32, iota)
    svb, spb = lax.sort_key_val(x32, iota)
    return _word2vec_dot(sva, spa, svb, spb, in_table.T, out_table.T)


# speedup vs baseline: 44.7155x; 1.0216x over previous
import functools
import jax
import jax.numpy as jnp
from jax import lax
from jax.experimental import pallas as pl
from jax.experimental.pallas import tpu as pltpu
from jax.experimental.pallas import tpu_sc as plsc

B = 16384
D = 64
V = 1000000
L = 16
BLK = 128
LAST_BLK = (V - 1) // BLK
WIN = 5
BIG = 1 << 30

_info = plsc.get_sparse_core_info()
NC = _info.num_cores
NS = _info.num_subcores
NW = NC * NS
B_PER_W = B // NW
GROUPS = B_PER_W // L

_PARAMS = pltpu.CompilerParams(needs_layout_passes=False)
_MESH = dict(core_axis_name="c", subcore_axis_name="s")


def _sweep_table(sv_hbm, sp_hbm, vt, stage_hbm, sv, sp, slab, colst, sem,
                 base, lane):
    pltpu.sync_copy(sv_hbm.at[pl.ds(base, B_PER_W)], sv.at[pl.ds(0, B_PER_W)])
    sv[pl.ds(B_PER_W, L)] = jnp.full((L,), BIG, jnp.int32)
    sv[pl.ds(B_PER_W + L, L)] = jnp.full((L,), BIG, jnp.int32)
    pltpu.sync_copy(sp_hbm.at[pl.ds(base, B_PER_W)], sp)

    blk_lo = lax.shift_right_logical(sv[pl.ds(0, L)][0], 7)
    blk_hi = lax.shift_right_logical(sv[pl.ds(B_PER_W - L, L)][L - 1], 7)
    nwin = lax.div(blk_hi - blk_lo, WIN) + 1
    nwp = lax.div(nwin + 1, 2)

    def fire(w, s):
        for b in range(WIN):
            blk = jnp.minimum(blk_lo + w * WIN + b, LAST_BLK)
            start = pl.multiple_of(blk * BLK, BLK)
            pltpu.async_copy(vt.at[:, pl.ds(start, BLK)], slab.at[s, b], sem)

    def drain(s):
        for b in range(WIN):
            pltpu.make_async_copy(
                vt.at[:, pl.ds(0, BLK)], slab.at[s, b], sem).wait()

    def process(w, s, p):
        wbase = blk_lo + w * WIN
        wend = (wbase + WIN) * BLK
        view = slab.at[s]

        def wcond(st):
            return st[1]

        def wbody(st):
            p, _ = st
            v16 = plsc.load_gather(sv, [p + lane])
            mask = v16 < wend
            cnt = plsc.all_reduce_population_count(mask)[0]

            @pl.when(cnt > 0)
            def _():
                q = lax.shift_right_logical(v16, 7) - wbase
                col = v16 & (BLK - 1)
                obase = (p + lane) * D
                for dd in range(D):
                    rows = (lane + dd) & (D - 1)
                    vals = plsc.load_gather(view, [q, rows, col], mask=mask)
                    plsc.store_scatter(colst, [obase + rows], vals, mask=mask)

            return p + cnt, cnt == L

        p, _ = lax.while_loop(wcond, wbody, (p, True))
        return p

    fire(0, 0)

    def body(wp, p):
        w0 = wp * 2
        fire(w0 + 1, 1)
        drain(0)
        p = process(w0, 0, p)
        fire(w0 + 2, 0)
        drain(1)
        p = process(w0 + 1, 1, p)
        return p

    lax.fori_loop(0, nwp, body, jnp.int32(0))
    drain(0)

    def stage(g, _):
        pos16 = sp[pl.ds(g * L, L)]
        copies = []
        for t in range(L):
            src = colst.at[pl.ds((g * L + t) * D, D)]
            dst = stage_hbm.at[pl.ds(pos16[t] * D, D)]
            copies.append(pltpu.async_copy(src, dst, sem))
        for cp in copies:
            cp.wait()
        return 0

    lax.fori_loop(0, GROUPS, stage, 0)


def _gather_kernel(sva, spa, svb, spb, vta, vtb, sa_hbm, sb_hbm,
                   sv, sp, slab, colst, sem):
    wid = lax.axis_index("s") * NC + lax.axis_index("c")
    base = wid * B_PER_W
    lane = lax.iota(jnp.int32, L)
    _sweep_table(sva, spa, vta, sa_hbm, sv, sp, slab, colst, sem, base, lane)
    _sweep_table(svb, spb, vtb, sb_hbm, sv, sp, slab, colst, sem, base, lane)


def _combine_kernel(sa_hbm, sb_hbm, dot_hbm, ga, gb, out_v, sem):
    wid = lax.axis_index("s") * NC + lax.axis_index("c")
    base = wid * B_PER_W
    lane = lax.iota(jnp.int32, L)

    ca = pltpu.async_copy(sa_hbm.at[pl.ds(base * D, B_PER_W * D)], ga, sem)
    cb = pltpu.async_copy(sb_hbm.at[pl.ds(base * D, B_PER_W * D)], gb, sem)
    ca.wait()
    cb.wait()

    def dots(g, _):
        obase = (g * L + lane) * D
        acc = None
        for dd in range(D):
            addr = obase + ((lane + dd) & (D - 1))
            prod = plsc.load_gather(ga, [addr]) * plsc.load_gather(gb, [addr])
            acc = prod if acc is None else acc + prod
        out_v[pl.ds(g * L, L)] = acc
        return 0

    lax.fori_loop(0, GROUPS, dots, 0)

    pltpu.sync_copy(out_v, dot_hbm.at[pl.ds(base, B_PER_W)])


@jax.jit
def _word2vec_dot(sva, spa, svb, spb, vta, vtb):
    gather = functools.partial(
        pl.kernel,
        out_type=(jax.ShapeDtypeStruct((B * D,), jnp.float32),
                  jax.ShapeDtypeStruct((B * D,), jnp.float32)),
        mesh=plsc.VectorSubcoreMesh(**_MESH),
        compiler_params=_PARAMS,
        scratch_types=[
            pltpu.VMEM((B_PER_W + 2 * L,), jnp.int32),
            pltpu.VMEM((B_PER_W,), jnp.int32),
            pltpu.VMEM((2, WIN, D, BLK), jnp.float32),
            pltpu.VMEM((B_PER_W * D,), jnp.float32),
            pltpu.SemaphoreType.DMA,
        ],
    )(_gather_kernel)
    sa, sb = gather(sva, spa, svb, spb, vta, vtb)

    combine = functools.partial(
        pl.kernel,
        out_type=jax.ShapeDtypeStruct((B,), jnp.float32),
        mesh=plsc.VectorSubcoreMesh(**_MESH),
        compiler_params=_PARAMS,
        scratch_types=[
            pltpu.VMEM((B_PER_W * D,), jnp.float32),
            pltpu.VMEM((B_PER_W * D,), jnp.float32),
            pltpu.VMEM((B_PER_W,), jnp.float32),
            pltpu.SemaphoreType.DMA,
        ],
    )(_combine_kernel)
    return combine(sa, sb)


def kernel(center, context, in_table, out_table):
    c32 = center.astype(jnp.int32)
    x32 = context.astype(jnp.int32)
    iota = jnp.arange(B, dtype=jnp.int32)
    sva, spa = lax.sort_key_val(c32, iota)
    svb, spb = lax.sort_key_val(x32, iota)
    return _word2vec_dot(sva, spa, svb, spb, in_table.T, out_table.T)
